# baseline (speedup 1.0000x reference)
import functools
import math

import jax
import jax.numpy as jnp
import numpy as np
from jax import lax
from jax.experimental import pallas as pl
from jax.experimental.pallas import tpu as pltpu
from jax.experimental.pallas import tpu_sc as plsc

NUM_NODES = 100000
SIZE = 10
HID = 64
TDIM = 64
DIMS = HID + TDIM
B = 16384

NW = 32
EPW = B // NW
NBUF = 16
NSTEP = EPW // NBUF


def _sc_gather(n_id, msg3d, t2d, msg_count, t_ref):
  mesh = plsc.VectorSubcoreMesh(core_axis_name="c", subcore_axis_name="s")

  @functools.partial(
      pl.kernel,
      mesh=mesh,
      compiler_params=pltpu.CompilerParams(needs_layout_passes=False),
      out_type=[
          jax.ShapeDtypeStruct((B, SIZE, HID), jnp.float32),
          jax.ShapeDtypeStruct((B, 32), jnp.float32),
      ],
      scratch_types=[
          pltpu.VMEM((NBUF, SIZE, HID), jnp.float32),
          pltpu.VMEM((EPW,), jnp.int32),
          pltpu.VMEM((EPW,), jnp.float32),
          pltpu.VMEM((128, 16), jnp.float32),
          pltpu.VMEM((128, 32), jnp.float32),
          pltpu.VMEM((EPW,), jnp.int32),
      ] + [pltpu.SemaphoreType.DMA] * (NBUF + 1),
  )
  def gather_kernel(nid_hbm, msg_hbm, t_hbm, mc_hbm, tref_hbm,
                    out_msg, out_meta,
                    mbuf, nid_v, trefv, t_e, dm_v, mcg_v,
                    *sems_all):
    sems = sems_all[:NBUF]
    sem_t = sems_all[NBUF]
    wid = lax.axis_index("s") * 2 + lax.axis_index("c")
    ebase = wid * EPW

    pltpu.sync_copy(nid_hbm.at[pl.ds(ebase, EPW)], nid_v)
    pltpu.sync_copy(tref_hbm.at[pl.ds(ebase, EPW)], trefv)
    mc_handles = [
        pltpu.async_copy(mc_hbm.at[nid_v.at[pl.ds(k * 128, 128)]],
                         mcg_v.at[pl.ds(k * 128, 128)], sem_t)
        for k in range(4)]
    for h in mc_handles:
      h.wait()

    lane = lax.iota(jnp.int32, 16)

    def extract_i(vec, l):
      return jnp.sum(jnp.where(lane == l, vec, 0))

    def extract_f(vec, l):
      return jnp.sum(jnp.where(lane == l, vec, 0.0))

    def ring_step(g, _):
      v16 = nid_v[pl.ds(g * 16, 16)]
      for l in range(NBUF):
        @pl.when(g > 0)
        def _():
          pltpu.make_async_copy(msg_hbm.at[0], mbuf.at[l], sems[l]).wait()
          pltpu.sync_copy(mbuf.at[l], out_msg.at[ebase + (g - 1) * 16 + l])
        pltpu.async_copy(msg_hbm.at[extract_i(v16, l)], mbuf.at[l], sems[l])
      return 0
    lax.fori_loop(0, EPW // 16, ring_step, 0)
    for l in range(NBUF):
      pltpu.make_async_copy(msg_hbm.at[0], mbuf.at[l], sems[l]).wait()
      pltpu.sync_copy(mbuf.at[l],
                      out_msg.at[ebase + (EPW // 16 - 1) * 16 + l])

    def meta_round(r, _):
      cbase = r * 128

      def t_fire(k, _):
        grp = (cbase + k) // 16
        v = nid_v[pl.ds(grp * 16, 16)]
        n = jnp.sum(jnp.where(lane == (cbase + k) % 16, v, 0))
        pltpu.async_copy(t_hbm.at[n], t_e.at[k, pl.ds(0, SIZE)], sem_t)
        return 0
      lax.fori_loop(0, 128, t_fire, 0)

      def t_drain(k, _):
        pltpu.make_async_copy(t_hbm.at[0], t_e.at[k, pl.ds(0, SIZE)],
                              sem_t).wait()
        return 0
      lax.fori_loop(0, 128, t_drain, 0)

      def meta_step(k, _):
        e = cbase + k
        grp = e // 16
        tr16 = trefv[pl.ds(grp * 16, 16)]
        mc16 = mcg_v[pl.ds(grp * 16, 16)]
        l = e % 16
        dt16 = extract_f(tr16, l) - t_e[k]
        msk16 = jnp.where(lane < extract_i(mc16, l), 1.0, 0.0)
        dm_v[k, pl.ds(0, 16)] = dt16
        dm_v[k, pl.ds(16, 16)] = msk16
        return 0
      lax.fori_loop(0, 128, meta_step, 0)

      pltpu.sync_copy(dm_v, out_meta.at[pl.ds(ebase + cbase, 128)])
      return 0
    lax.fori_loop(0, 4, meta_round, 0)

  return gather_kernel(n_id, msg3d, t2d, msg_count, t_ref)


BB = 256
RR = BB * SIZE

_COS_C = [0.125 * c for c in
          (1.0, -19.739208, 64.93939, -85.45669, 60.242466,
           -26.406763, 7.8066154, -1.4609568)]
_INV_2PI = 1.0 / (2.0 * math.pi)

_SEL = (np.repeat(np.eye(BB, dtype=np.float32), SIZE, axis=1) / SIZE)
_EROWS = np.repeat(np.eye(BB, dtype=np.float32), SIZE, axis=0)
_r = np.arange(RR) % SIZE
_OH_DT = (np.arange(32)[None, :] == _r[:, None]).astype(np.float32)
_OH_MSK = (np.arange(32)[None, :] == 16 + _r[:, None]).astype(np.float32)
_LANEOK = ((np.arange(32) < SIZE) | ((np.arange(32) >= 16) &
                                     (np.arange(32) < 16 + SIZE))
           ).astype(np.float32).reshape(1, 32)


def _tc_mixer(msg_rows, meta, freq_row, tW_t, tb, cW_t, cb,
              tg, tbeta, cg, cbeta):
  grid = (B // BB,)

  def body(msg_ref, meta_ref, freq_ref, ohdt_ref, ohmsk_ref,
           laneok_ref, sel_ref,
           tw_ref, tb_ref, cw_ref, cb_ref,
           tg_ref, tbt_ref, cg_ref, cbt_ref, out_ref):
    meta_clean = meta_ref[...] * laneok_ref[...]
    tmp = jnp.repeat(meta_clean, SIZE, axis=0)
    dt = jnp.sum(tmp * ohdt_ref[...], axis=1, keepdims=True)
    mask = jnp.sum(tmp * ohmsk_ref[...], axis=1, keepdims=True)

    y = (dt * freq_ref[...]) * _INV_2PI
    y = y - lax.round(y, lax.RoundingMethod.TO_NEAREST_EVEN)
    u = y * y
    enc = _COS_C[7]
    for k in range(6, -1, -1):
      enc = enc * u + _COS_C[k]
    x = jnp.concatenate([enc, msg_ref[...]], axis=1) * mask

    def ln(v, g, b):
      mu = jnp.mean(v, axis=1, keepdims=True)
      var = jnp.mean((v - mu) ** 2, axis=1, keepdims=True)
      return (v - mu) * lax.rsqrt(var + 1e-5) * g + b

    def gelu(v):
      return 0.5 * v * (1.0 + lax.erf(v * (1.0 / math.sqrt(2.0))))

    h = ln(x, tg_ref[...], tbt_ref[...])
    h = gelu(jnp.dot(h, tw_ref[...], preferred_element_type=jnp.float32)
             + tb_ref[...])
    x = x + h
    h = ln(x, cg_ref[...], cbt_ref[...])
    h = gelu(jnp.dot(h, cw_ref[...], preferred_element_type=jnp.float32)
             + cb_ref[...])
    x = x + h

    out_ref[...] = jnp.dot(sel_ref[...], x, preferred_element_type=jnp.float32)

  full = lambda shp: pl.BlockSpec(shp, lambda i: (0, 0))
  return pl.pallas_call(
      body,
      grid=grid,
      in_specs=[
          pl.BlockSpec((RR, HID), lambda i: (i, 0)),
          pl.BlockSpec((BB, 32), lambda i: (i, 0)),
          full((1, TDIM)),
          full((RR, 32)), full((RR, 32)), full((1, 32)),
          full((BB, RR)),
          full((DIMS, DIMS)), full((1, DIMS)),
          full((DIMS, DIMS)), full((1, DIMS)),
          full((1, DIMS)), full((1, DIMS)), full((1, DIMS)), full((1, DIMS)),
      ],
      out_specs=pl.BlockSpec((BB, DIMS), lambda i: (i, 0)),
      out_shape=jax.ShapeDtypeStruct((B, DIMS), jnp.float32),
  )(msg_rows, meta, freq_row,
    jnp.asarray(_OH_DT), jnp.asarray(_OH_MSK),
    jnp.asarray(_LANEOK), jnp.asarray(_SEL),
    tW_t, tb, cW_t, cb, tg, tbeta, cg, cbeta)


def kernel(n_id, t_ref, msg_store, t_store, msg_count,
           token_gamma, token_beta, token_W, token_b,
           chan_gamma, chan_beta, chan_W, chan_b):
  msg_g, meta = _sc_gather(n_id.astype(jnp.int32), msg_store, t_store,
                           msg_count, t_ref)
  msg_rows = msg_g.reshape(B * SIZE, HID)

  freq_row = (1.0 / (10.0 ** jnp.linspace(0.0, 9.0, TDIM,
                                          dtype=jnp.float32))).reshape(1, TDIM)

  return _tc_mixer(msg_rows, meta, freq_row,
                   token_W.T, token_b.reshape(1, DIMS),
                   chan_W.T, chan_b.reshape(1, DIMS),
                   token_gamma.reshape(1, DIMS), token_beta.reshape(1, DIMS),
                   chan_gamma.reshape(1, DIMS), chan_beta.reshape(1, DIMS))


# speedup vs baseline: 1.1671x; 1.1671x over previous
import functools
import math

import jax
import jax.numpy as jnp
import numpy as np
from jax import lax
from jax.experimental import pallas as pl
from jax.experimental.pallas import tpu as pltpu
from jax.experimental.pallas import tpu_sc as plsc

NUM_NODES = 100000
SIZE = 10
HID = 64
TDIM = 64
DIMS = HID + TDIM
B = 16384

NW = 32
CH = 64
EPW = B // NW
NCHUNK = EPW // CH


def _sc_gather(nid2d, msg2d, t_flat, msg_count):
  mesh = plsc.VectorSubcoreMesh(core_axis_name="c", subcore_axis_name="s")

  @functools.partial(
      pl.kernel,
      mesh=mesh,
      compiler_params=pltpu.CompilerParams(use_tc_tiling_on_sc=False),
      out_type=[
          jax.ShapeDtypeStruct((B, SIZE * HID), jnp.float32),
          jax.ShapeDtypeStruct((NW, SIZE, EPW), jnp.float32),
          jax.ShapeDtypeStruct((B,), jnp.int32),
      ],
      scratch_types=[
          pltpu.VMEM((NCHUNK, CH), jnp.int32),
          pltpu.VMEM((CH, SIZE * HID), jnp.float32),
          pltpu.VMEM((CH, SIZE * HID), jnp.float32),
          pltpu.VMEM((SIZE, EPW), jnp.float32),
          pltpu.VMEM((NCHUNK * SIZE, CH), jnp.int32),
          pltpu.VMEM((EPW,), jnp.int32),
          pltpu.SemaphoreType.DMA,
          pltpu.SemaphoreType.DMA,
          pltpu.SemaphoreType.DMA,
      ],
  )
  def gather_kernel(nid_hbm, msg_hbm, t_hbm, mc_hbm,
                    out_msg, out_t, out_mc,
                    idx_v, buf_a, buf_b, t_v, tidx_v, mc_v,
                    sem_a, sem_b, sem_small):
    wid = lax.axis_index("s") * 2 + lax.axis_index("c")
    rbase = wid * NCHUNK
    ebase = wid * EPW

    pltpu.sync_copy(nid_hbm.at[pl.ds(rbase, NCHUNK)], idx_v)

    handles = []
    for j in range(NCHUNK):
      for g in range(CH // 16):
        v = idx_v[j, pl.ds(g * 16, 16)] * SIZE
        for s in range(SIZE):
          tidx_v[j * SIZE + s, pl.ds(g * 16, 16)] = v + s
      for s in range(SIZE):
        handles.append(pltpu.async_copy(
            t_hbm.at[tidx_v.at[j * SIZE + s]],
            t_v.at[s, pl.ds(j * CH, CH)], sem_small))
      handles.append(pltpu.async_copy(
          mc_hbm.at[idx_v.at[j]], mc_v.at[pl.ds(j * CH, CH)], sem_small))
    for h in handles:
      h.wait()
    pltpu.sync_copy(t_v, out_t.at[wid])
    pltpu.sync_copy(mc_v, out_mc.at[pl.ds(ebase, EPW)])

    bufs = (buf_a, buf_b)
    sems = (sem_a, sem_b)
    prev = pltpu.async_copy(msg_hbm.at[idx_v.at[0]], bufs[0], sems[0])
    for j in range(1, NCHUNK):
      cur = pltpu.async_copy(msg_hbm.at[idx_v.at[j]], bufs[j % 2],
                             sems[j % 2])
      prev.wait()
      pltpu.sync_copy(bufs[(j - 1) % 2],
                      out_msg.at[pl.ds(ebase + (j - 1) * CH, CH)])
      prev = cur
    prev.wait()
    pltpu.sync_copy(bufs[(NCHUNK - 1) % 2],
                    out_msg.at[pl.ds(ebase + (NCHUNK - 1) * CH, CH)])

  return gather_kernel(nid2d, msg2d, t_flat, msg_count)


BB = 256
RR = BB * SIZE

_COS_C = [0.125 * c for c in
          (1.0, -19.739208, 64.93939, -85.45669, 60.242466,
           -26.406763, 7.8066154, -1.4609568)]
_INV_2PI = 1.0 / (2.0 * math.pi)

_SLOT_COL = np.tile(np.arange(SIZE, dtype=np.float32), BB).reshape(BB * SIZE, 1)
_SEL = (np.repeat(np.eye(BB, dtype=np.float32), SIZE, axis=1) / SIZE)


def _tc_mixer(msg_rows, t_rows, tref_rows, mc_rows, freq_row,
              tW_t, tb, cW_t, cb, tg, tbeta, cg, cbeta):
  grid = (B // BB,)

  def body(msg_ref, t_ref, tr_ref, mc_ref, freq_ref, slot_ref, sel_ref,
           tw_ref, tb_ref, cw_ref, cb_ref,
           tg_ref, tbt_ref, cg_ref, cbt_ref, out_ref):
    dt = tr_ref[...] - t_ref[...]
    y = (dt * freq_ref[...]) * _INV_2PI
    y = y - lax.round(y, lax.RoundingMethod.TO_NEAREST_EVEN)
    u = y * y
    enc = _COS_C[7]
    for k in range(6, -1, -1):
      enc = enc * u + _COS_C[k]
    mask = (slot_ref[...] < mc_ref[...]).astype(jnp.float32)
    x = jnp.concatenate([enc, msg_ref[...]], axis=1) * mask

    def ln(v, g, b):
      mu = jnp.mean(v, axis=1, keepdims=True)
      var = jnp.mean((v - mu) ** 2, axis=1, keepdims=True)
      return (v - mu) * lax.rsqrt(var + 1e-5) * g + b

    def gelu(v):
      return 0.5 * v * (1.0 + lax.erf(v * (1.0 / math.sqrt(2.0))))

    h = ln(x, tg_ref[...], tbt_ref[...])
    h = gelu(jnp.dot(h, tw_ref[...], preferred_element_type=jnp.float32)
             + tb_ref[...])
    x = x + h
    h = ln(x, cg_ref[...], cbt_ref[...])
    h = gelu(jnp.dot(h, cw_ref[...], preferred_element_type=jnp.float32)
             + cb_ref[...])
    x = x + h

    out_ref[...] = jnp.dot(sel_ref[...], x, preferred_element_type=jnp.float32)

  col = pl.BlockSpec((RR, 1), lambda i: (i, 0))
  full = lambda shp: pl.BlockSpec(shp, lambda i: (0, 0))
  return pl.pallas_call(
      body,
      grid=grid,
      in_specs=[
          pl.BlockSpec((RR, HID), lambda i: (i, 0)),
          col, col, col,
          full((1, TDIM)),
          full((RR, 1)), full((BB, RR)),
          full((DIMS, DIMS)), full((1, DIMS)),
          full((DIMS, DIMS)), full((1, DIMS)),
          full((1, DIMS)), full((1, DIMS)), full((1, DIMS)), full((1, DIMS)),
      ],
      out_specs=pl.BlockSpec((BB, DIMS), lambda i: (i, 0)),
      out_shape=jax.ShapeDtypeStruct((B, DIMS), jnp.float32),
  )(msg_rows, t_rows, tref_rows, mc_rows, freq_row,
    jnp.asarray(_SLOT_COL), jnp.asarray(_SEL),
    tW_t, tb, cW_t, cb, tg, tbeta, cg, cbeta)


def kernel(n_id, t_ref, msg_store, t_store, msg_count,
           token_gamma, token_beta, token_W, token_b,
           chan_gamma, chan_beta, chan_W, chan_b):
  nid2d = n_id.astype(jnp.int32).reshape(B // CH, CH)
  msg2d = msg_store.reshape(NUM_NODES, SIZE * HID)

  msg_g, t_g, mc_g = _sc_gather(nid2d, msg2d, t_store.reshape(-1), msg_count)

  msg_rows = msg_g.reshape(B * SIZE, HID)
  t_rows = jnp.transpose(t_g, (0, 2, 1)).reshape(B * SIZE, 1)
  tref_rows = jnp.repeat(t_ref, SIZE).reshape(B * SIZE, 1)
  mc_rows = jnp.repeat(mc_g.astype(jnp.float32), SIZE).reshape(B * SIZE, 1)

  freq_row = (1.0 / (10.0 ** jnp.linspace(0.0, 9.0, TDIM,
                                          dtype=jnp.float32))).reshape(1, TDIM)

  return _tc_mixer(msg_rows, t_rows, tref_rows, mc_rows, freq_row,
                   token_W.T, token_b.reshape(1, DIMS),
                   chan_W.T, chan_b.reshape(1, DIMS),
                   token_gamma.reshape(1, DIMS), token_beta.reshape(1, DIMS),
                   chan_gamma.reshape(1, DIMS), chan_beta.reshape(1, DIMS))
